# baseline (speedup 1.0000x reference)
import functools
import math

import jax
import jax.numpy as jnp
from jax import lax
from jax.experimental import pallas as pl
from jax.experimental.pallas import tpu as pltpu
from jax.experimental.pallas import tpu_sc as plsc

V = 100000
T = 10000
B = 4096
L = 50
D = 128
TE = 16
K = 16

NW = 32
CH = 128
NIDS = B * (L + 1)
N_PER = NIDS // NW
NCH = N_PER // CH
FW = D // 2
NBUF = 4


def _sc_gather_body(ids_all, fro_tab, tags16, tag_tab, fro_all, te_all,
                    *scr):
    bufs = [scr[i * 6:(i + 1) * 6] for i in range(NBUF)]
    sems = scr[NBUF * 6:]
    wid = lax.axis_index("s") * 2 + lax.axis_index("c")
    wbase = wid * N_PER
    lanes = lax.iota(jnp.int32, 16)
    pat128 = lanes * CH
    mask5 = lanes < 5

    def sem4(k, j):
        return sems[k * 4 + j]

    def issue(c, k):
        idx_v, tg_v, slot_v, fro_v, tr_v, te_v = bufs[k]
        static = isinstance(c, int)
        base = wbase + (min(c, NCH - 1) if static else c) * CH

        def drain():
            pltpu.make_async_copy(fro_v, fro_all.at[pl.ds(base, CH)],
                                  sem4(k, 3)).wait()
            pltpu.make_async_copy(te_v, te_all.at[pl.ds(base, CH)],
                                  sem4(k, 3)).wait()

        def load():
            pltpu.sync_copy(ids_all.at[pl.ds(base, CH)], idx_v)
            pltpu.async_copy(fro_tab.at[idx_v], fro_v, sem4(k, 0))
            pltpu.async_copy(tags16.at[idx_v], tg_v, sem4(k, 1))

        if static:
            if c >= NBUF:
                drain()
            if c < NCH:
                load()
        else:
            pl.when(c >= NBUF)(drain)
            load()

    def process(c, k):
        idx_v, tg_v, slot_v, fro_v, tr_v, te_v = bufs[k]
        base = wbase + c * CH
        pltpu.make_async_copy(tags16.at[idx_v], tg_v, sem4(k, 1)).wait()

        def tr_body(i, _):
            plsc.store_scatter(slot_v, [pat128 + i], tg_v[i, :], mask=mask5)
            return 0

        lax.fori_loop(0, CH, tr_body, 0)
        h_t = [pltpu.async_copy(tag_tab.at[slot_v.at[pl.ds(s * CH, CH)]],
                                tr_v.at[s], sem4(k, 2))
               for s in range(5)]
        for h in h_t:
            h.wait()

        def sum_body(i, _):
            te_v[i, :] = (tr_v[0, i, :] + tr_v[1, i, :] + tr_v[2, i, :]
                          + tr_v[3, i, :] + tr_v[4, i, :])
            return 0

        lax.fori_loop(0, CH, sum_body, 0)
        pltpu.make_async_copy(fro_tab.at[idx_v], fro_v, sem4(k, 0)).wait()
        pltpu.async_copy(fro_v, fro_all.at[pl.ds(base, CH)], sem4(k, 3))
        pltpu.async_copy(te_v, te_all.at[pl.ds(base, CH)], sem4(k, 3))

    issue(0, 0)
    issue(1, 1)

    def step(c, k):
        issue(c + 2, (k + 2) % NBUF)
        process(c, k)

    def body(g, _):
        c = g * NBUF
        for k in range(NBUF):
            step(c + k, k)
        return 0

    nfull = (NCH // NBUF) * NBUF
    lax.fori_loop(0, NCH // NBUF, body, 0)
    for k in range(NCH - nfull):
        step(nfull + k, k)
    for c in range(NCH - 2, NCH):
        k = c % NBUF
        idx_v, tg_v, slot_v, fro_v, tr_v, te_v = bufs[k]
        base = wbase + c * CH
        pltpu.make_async_copy(fro_v, fro_all.at[pl.ds(base, CH)],
                              sem4(k, 3)).wait()
        pltpu.make_async_copy(te_v, te_all.at[pl.ds(base, CH)],
                              sem4(k, 3)).wait()


def _sc_gather(ids_all, fro_tab, tags16, tag_tab):
    mesh = plsc.VectorSubcoreMesh(core_axis_name="c", subcore_axis_name="s")
    scratch = []
    for _ in range(NBUF):
        scratch += [
            pltpu.VMEM((CH,), jnp.int32),
            pltpu.VMEM((CH, 16), jnp.int32),
            pltpu.VMEM((5 * CH,), jnp.int32),
            pltpu.VMEM((CH, FW), jnp.int32),
            pltpu.VMEM((5, CH, TE), jnp.float32),
            pltpu.VMEM((CH, TE), jnp.float32),
        ]
    scratch += [pltpu.SemaphoreType.DMA] * (NBUF * 4)
    f = pl.kernel(
        _sc_gather_body,
        out_type=(
            jax.ShapeDtypeStruct((NIDS, FW), jnp.int32),
            jax.ShapeDtypeStruct((NIDS, TE), jnp.float32),
        ),
        mesh=mesh,
        compiler_params=pltpu.CompilerParams(use_tc_tiling_on_sc=False,
                                             needs_layout_passes=False),
        scratch_types=scratch,
    )
    return f(ids_all, fro_tab, tags16, tag_tab)


def _tc_body(fro_s_ref, te_s_ref, fro_t_ref, te_t_ref, seqid_ref,
             likes_ref, views_ref,
             wfa_ref, wfb_ref, bf_ref, wat_ref, wtp_ref, btp_ref,
             ltab_ref, vtab_ref,
             wc1_ref, bc1_ref, wc2_ref, bc2_ref,
             wd1_ref, bd1_ref, wd2_ref, bd2_ref,
             woa_ref, wob_ref, bo_ref, out_ref, *, bb):
    def bfc(x):
        return x.astype(jnp.bfloat16)

    def dotf(a, b):
        return jnp.dot(bfc(a), bfc(b), preferred_element_type=jnp.float32)

    wfa = wfa_ref[...]
    wfb = wfb_ref[...]
    bf = bf_ref[...]

    s_in = fro_s_ref[...]
    te_s = te_s_ref[...] * 0.2
    s_emb = jnp.tanh(dotf(s_in, wfa) + dotf(te_s, wfb) + bf)
    t_emb = jnp.tanh(dotf(fro_t_ref[...], wfa)
                     + dotf(te_t_ref[...] * 0.2, wfb) + bf)

    q = dotf(t_emb, wat_ref[...])
    s3 = s_emb.reshape(bb, L, 128)
    s3b = bfc(s3).astype(jnp.float32)
    scores = jnp.sum(s3b * bfc(q)[:, None, :].astype(jnp.float32), axis=-1)
    scores = scores * (1.0 / math.sqrt(128.0))
    scores = jnp.where(seqid_ref[...] == 0, -1e9, scores)

    lp = lax.broadcasted_iota(jnp.int32, (bb, L, L), 1)
    ll = lax.broadcasted_iota(jnp.int32, (bb, L, L), 2)
    a_lp = scores[:, :, None]
    a_l = scores[:, None, :]
    rank = jnp.sum(
        (a_lp > a_l).astype(jnp.int32)
        + jnp.logical_and(a_lp == a_l, lp < ll).astype(jnp.int32), axis=1)
    sel = rank < K

    m = jnp.max(scores, axis=1, keepdims=True)
    e = jnp.where(sel, jnp.exp(scores - m), 0.0)
    w = e / jnp.sum(e, axis=1, keepdims=True)
    wb = bfc(w).astype(jnp.float32)
    S_o = jnp.sum(wb[:, :, None] * s3b, axis=1)

    t_proj = dotf(t_emb, wtp_ref[...]) + btp_ref[...]
    lv_iota = lax.broadcasted_iota(jnp.int32, (bb, 10), 1)
    l_oh = (likes_ref[...] == lv_iota).astype(jnp.float32)
    v_oh = (views_ref[...] == lv_iota).astype(jnp.float32)
    le = jnp.sum(l_oh[:, :, None] * ltab_ref[...][None, :, :], axis=1)
    ve = jnp.sum(v_oh[:, :, None] * vtab_ref[...][None, :, :], axis=1)

    x0 = jnp.concatenate([S_o, t_proj, le, ve], axis=-1)
    x1 = x0 * (dotf(x0, wc1_ref[...]) + bc1_ref[...]) + x0
    x2 = x0 * (dotf(x1, wc2_ref[...]) + bc2_ref[...]) + x1
    deep = jnp.maximum(dotf(x0, wd1_ref[...]) + bd1_ref[...], 0.0)
    deep = jnp.maximum(dotf(deep, wd2_ref[...]) + bd2_ref[...], 0.0)
    out_ref[...] = (dotf(x2, woa_ref[...]) + dotf(deep, wob_ref[...])
                    + bo_ref[...])


def _tc_call(fro_seq, te_seq, fro_tgt, te_tgt, item_seqs, likes2, views2,
             wfa, wfb, bf2, wat, wtp, btp2, ltab, vtab,
             wc1, bc12, wc2, bc22, wd1, bd12, wd2, bd22, woa, wob, bo2):
    bb = 256
    grid = (B // bb,)

    full0 = lambda i: (0, 0)
    specs = [
        pl.BlockSpec((bb * L, D), lambda i: (i, 0)),
        pl.BlockSpec((bb * L, TE), lambda i: (i, 0)),
        pl.BlockSpec((bb, D), lambda i: (B * L // bb + i, 0)),
        pl.BlockSpec((bb, TE), lambda i: (B * L // bb + i, 0)),
        pl.BlockSpec((bb, L), lambda i: (i, 0)),
        pl.BlockSpec((bb, 1), lambda i: (i, 0)),
        pl.BlockSpec((bb, 1), lambda i: (i, 0)),
        pl.BlockSpec((D, 128), full0),
        pl.BlockSpec((TE, 128), full0),
        pl.BlockSpec((1, 128), full0),
        pl.BlockSpec((128, 128), full0),
        pl.BlockSpec((128, 192), full0),
        pl.BlockSpec((1, 192), full0),
        pl.BlockSpec((10, 16), full0),
        pl.BlockSpec((10, 16), full0),
        pl.BlockSpec((352, 352), full0),
        pl.BlockSpec((1, 352), full0),
        pl.BlockSpec((352, 352), full0),
        pl.BlockSpec((1, 352), full0),
        pl.BlockSpec((352, 256), full0),
        pl.BlockSpec((1, 256), full0),
        pl.BlockSpec((256, 128), full0),
        pl.BlockSpec((1, 128), full0),
        pl.BlockSpec((352, 1), full0),
        pl.BlockSpec((128, 1), full0),
        pl.BlockSpec((1, 1), full0),
    ]
    return pl.pallas_call(
        functools.partial(_tc_body, bb=bb),
        grid=grid,
        in_specs=specs,
        out_specs=pl.BlockSpec((bb, 1), lambda i: (i, 0)),
        out_shape=jax.ShapeDtypeStruct((B, 1), jnp.float32),
    )(fro_seq, te_seq, fro_tgt, te_tgt, item_seqs, likes2, views2,
      wfa, wfb, bf2, wat, wtp, btp2, ltab, vtab,
      wc1, bc12, wc2, bc22, wd1, bd12, wd2, bd22, woa, wob, bo2)


def kernel(item_seqs, item_ids, likes_levels, views_levels, frozen_table,
           item_tags, tag_table, W_fuse, b_fuse, W_attn, W_tproj, b_tproj,
           likes_table, views_table, W_c1, b_c1, W_c2, b_c2, W_d1, b_d1,
           W_d2, b_d2, W_out, b_out):
    ids_all = jnp.concatenate([item_seqs.reshape(-1).astype(jnp.int32),
                               item_ids.astype(jnp.int32)])
    tags16 = jnp.pad(item_tags.astype(jnp.int32), ((0, 0), (0, 11)))
    fro_pack = jax.lax.bitcast_convert_type(
        frozen_table.astype(jnp.bfloat16).reshape(V, FW, 2), jnp.int32)

    fro_all, te_all = _sc_gather(ids_all, fro_pack, tags16, tag_table)
    fro_bf = jax.lax.bitcast_convert_type(
        fro_all, jnp.bfloat16).reshape(NIDS, D)
    fro_seq, fro_tgt = fro_bf, fro_bf
    te_seq, te_tgt = te_all, te_all

    wfa = W_fuse[:D]
    wfb = W_fuse[D:]
    wat = W_attn.T
    out = _tc_call(
        fro_seq, te_seq, fro_tgt, te_tgt, item_seqs.astype(jnp.int32),
        likes_levels.astype(jnp.int32).reshape(B, 1),
        views_levels.astype(jnp.int32).reshape(B, 1),
        wfa, wfb, b_fuse.reshape(1, 128), wat, W_tproj,
        b_tproj.reshape(1, 192), likes_table, views_table,
        W_c1, b_c1.reshape(1, 352), W_c2, b_c2.reshape(1, 352),
        W_d1, b_d1.reshape(1, 256), W_d2, b_d2.reshape(1, 128),
        W_out[:352], W_out[352:], b_out.reshape(1, 1))
    return out[:, 0]


# speedup vs baseline: 1.7867x; 1.7867x over previous
import functools
import math

import jax
import jax.numpy as jnp
from jax import lax
from jax.experimental import pallas as pl
from jax.experimental.pallas import tpu as pltpu
from jax.experimental.pallas import tpu_sc as plsc

V = 100000
T = 10000
B = 4096
L = 50
D = 128
TE = 16
K = 16

NW = 32
CH = 128
NIDS = B * (L + 1)
N_PER = NIDS // NW
NCH = N_PER // CH
FW = D // 2
NBUF = 4


def _sc_gather_body(seq_ids, tgt_ids, fro_tab, tags16, tag_tab,
                    fro_all, te_all, *scr):
    bufs = [scr[i * 6:(i + 1) * 6] for i in range(NBUF)]
    sems = scr[NBUF * 6:]
    wid = lax.axis_index("s") * 2 + lax.axis_index("c")
    wbase = wid * N_PER
    lanes = lax.iota(jnp.int32, 16)
    pat128 = lanes * CH
    mask5 = lanes < 5

    def sem4(k, j):
        return sems[k * 4 + j]

    def issue(c, k):
        idx_v, tg_v, slot_v, fro_v, tr_v, te_v = bufs[k]
        static = isinstance(c, int)
        base = wbase + (min(c, NCH - 1) if static else c) * CH

        def drain():
            pltpu.make_async_copy(fro_v, fro_all.at[pl.ds(base, CH)],
                                  sem4(k, 3)).wait()
            pltpu.make_async_copy(te_v, te_all.at[pl.ds(base, CH)],
                                  sem4(k, 3)).wait()

        def load():
            @pl.when(base < B * L)
            def _():
                pltpu.sync_copy(seq_ids.at[pl.ds(base, CH)], idx_v)

            @pl.when(base >= B * L)
            def _():
                pltpu.sync_copy(tgt_ids.at[pl.ds(base - B * L, CH)], idx_v)

            pltpu.async_copy(fro_tab.at[idx_v], fro_v, sem4(k, 0))
            pltpu.async_copy(tags16.at[idx_v], tg_v, sem4(k, 1))

        if static:
            if c >= NBUF:
                drain()
            if c < NCH:
                load()
        else:
            pl.when(c >= NBUF)(drain)
            load()

    def process(c, k):
        idx_v, tg_v, slot_v, fro_v, tr_v, te_v = bufs[k]
        base = wbase + c * CH
        pltpu.make_async_copy(tags16.at[idx_v], tg_v, sem4(k, 1)).wait()

        def tr_body(i, _):
            plsc.store_scatter(slot_v, [pat128 + i], tg_v[i, :], mask=mask5)
            return 0

        lax.fori_loop(0, CH, tr_body, 0)
        h_t = [pltpu.async_copy(tag_tab.at[slot_v.at[pl.ds(s * CH, CH)]],
                                tr_v.at[s], sem4(k, 2))
               for s in range(5)]
        for h in h_t:
            h.wait()

        def sum_body(i, _):
            te_v[i, :] = (tr_v[0, i, :] + tr_v[1, i, :] + tr_v[2, i, :]
                          + tr_v[3, i, :] + tr_v[4, i, :])
            return 0

        lax.fori_loop(0, CH, sum_body, 0)
        pltpu.make_async_copy(fro_tab.at[idx_v], fro_v, sem4(k, 0)).wait()
        pltpu.async_copy(fro_v, fro_all.at[pl.ds(base, CH)], sem4(k, 3))
        pltpu.async_copy(te_v, te_all.at[pl.ds(base, CH)], sem4(k, 3))

    issue(0, 0)
    issue(1, 1)

    def step(c, k):
        issue(c + 2, (k + 2) % NBUF)
        process(c, k)

    def body(g, _):
        c = g * NBUF
        for k in range(NBUF):
            step(c + k, k)
        return 0

    nfull = (NCH // NBUF) * NBUF
    lax.fori_loop(0, NCH // NBUF, body, 0)
    for k in range(NCH - nfull):
        step(nfull + k, k)
    for c in range(NCH - 2, NCH):
        k = c % NBUF
        idx_v, tg_v, slot_v, fro_v, tr_v, te_v = bufs[k]
        base = wbase + c * CH
        pltpu.make_async_copy(fro_v, fro_all.at[pl.ds(base, CH)],
                              sem4(k, 3)).wait()
        pltpu.make_async_copy(te_v, te_all.at[pl.ds(base, CH)],
                              sem4(k, 3)).wait()


def _sc_gather(seq_ids, tgt_ids, fro_tab, tags16, tag_tab):
    mesh = plsc.VectorSubcoreMesh(core_axis_name="c", subcore_axis_name="s")
    scratch = []
    for _ in range(NBUF):
        scratch += [
            pltpu.VMEM((CH,), jnp.int32),
            pltpu.VMEM((CH, 16), jnp.int32),
            pltpu.VMEM((5 * CH,), jnp.int32),
            pltpu.VMEM((CH, D), jnp.bfloat16),
            pltpu.VMEM((5, CH, TE), jnp.float32),
            pltpu.VMEM((CH, TE), jnp.float32),
        ]
    scratch += [pltpu.SemaphoreType.DMA] * (NBUF * 4)
    f = pl.kernel(
        _sc_gather_body,
        out_type=(
            jax.ShapeDtypeStruct((NIDS, D), jnp.bfloat16),
            jax.ShapeDtypeStruct((NIDS, TE), jnp.float32),
        ),
        mesh=mesh,
        compiler_params=pltpu.CompilerParams(use_tc_tiling_on_sc=False,
                                             needs_layout_passes=False),
        scratch_types=scratch,
    )
    return f(seq_ids, tgt_ids, fro_tab, tags16, tag_tab)


def _tc_body(fro_s_ref, te_s_ref, fro_t_ref, te_t_ref, seqid_ref,
             likes_ref, views_ref,
             wfa_ref, wfb_ref, bf_ref, wat_ref, wtp_ref, btp_ref,
             ltab_ref, vtab_ref,
             wc1_ref, bc1_ref, wc2_ref, bc2_ref,
             wd1_ref, bd1_ref, wd2_ref, bd2_ref,
             woa_ref, wob_ref, bo_ref, out_ref, *, bb):
    def bfc(x):
        return x.astype(jnp.bfloat16)

    def dotf(a, b):
        return jnp.dot(bfc(a), bfc(b), preferred_element_type=jnp.float32)

    wfa = wfa_ref[...]
    wfb = wfb_ref[...]
    bf = bf_ref[...]

    s_in = fro_s_ref[...]
    te_s = te_s_ref[...] * 0.2
    s_emb = jnp.tanh(dotf(s_in, wfa) + dotf(te_s, wfb) + bf)
    t_emb = jnp.tanh(dotf(fro_t_ref[...], wfa)
                     + dotf(te_t_ref[...] * 0.2, wfb) + bf)

    q = dotf(t_emb, wat_ref[...])
    s3 = s_emb.reshape(bb, L, 128)
    s3b = bfc(s3).astype(jnp.float32)
    scores = jnp.sum(s3b * bfc(q)[:, None, :].astype(jnp.float32), axis=-1)
    scores = scores * (1.0 / math.sqrt(128.0))
    scores = jnp.where(seqid_ref[...] == 0, -1e9, scores)

    lp = lax.broadcasted_iota(jnp.int32, (bb, L, L), 1)
    ll = lax.broadcasted_iota(jnp.int32, (bb, L, L), 2)
    a_lp = scores[:, :, None]
    a_l = scores[:, None, :]
    rank = jnp.sum(
        (a_lp > a_l).astype(jnp.int32)
        + jnp.logical_and(a_lp == a_l, lp < ll).astype(jnp.int32), axis=1)
    sel = rank < K

    m = jnp.max(scores, axis=1, keepdims=True)
    e = jnp.where(sel, jnp.exp(scores - m), 0.0)
    w = e / jnp.sum(e, axis=1, keepdims=True)
    wb = bfc(w).astype(jnp.float32)
    S_o = jnp.sum(wb[:, :, None] * s3b, axis=1)

    t_proj = dotf(t_emb, wtp_ref[...]) + btp_ref[...]
    lv_iota = lax.broadcasted_iota(jnp.int32, (bb, 10), 1)
    l_oh = (likes_ref[...] == lv_iota).astype(jnp.float32)
    v_oh = (views_ref[...] == lv_iota).astype(jnp.float32)
    le = jnp.sum(l_oh[:, :, None] * ltab_ref[...][None, :, :], axis=1)
    ve = jnp.sum(v_oh[:, :, None] * vtab_ref[...][None, :, :], axis=1)

    x0 = jnp.concatenate([S_o, t_proj, le, ve], axis=-1)
    x1 = x0 * (dotf(x0, wc1_ref[...]) + bc1_ref[...]) + x0
    x2 = x0 * (dotf(x1, wc2_ref[...]) + bc2_ref[...]) + x1
    deep = jnp.maximum(dotf(x0, wd1_ref[...]) + bd1_ref[...], 0.0)
    deep = jnp.maximum(dotf(deep, wd2_ref[...]) + bd2_ref[...], 0.0)
    out_ref[...] = (dotf(x2, woa_ref[...]) + dotf(deep, wob_ref[...])
                    + bo_ref[...])


def _tc_call(fro_seq, te_seq, fro_tgt, te_tgt, item_seqs, likes2, views2,
             wfa, wfb, bf2, wat, wtp, btp2, ltab, vtab,
             wc1, bc12, wc2, bc22, wd1, bd12, wd2, bd22, woa, wob, bo2):
    bb = 256
    grid = (B // bb,)

    full0 = lambda i: (0, 0)
    specs = [
        pl.BlockSpec((bb * L, D), lambda i: (i, 0)),
        pl.BlockSpec((bb * L, TE), lambda i: (i, 0)),
        pl.BlockSpec((bb, D), lambda i: (B * L // bb + i, 0)),
        pl.BlockSpec((bb, TE), lambda i: (B * L // bb + i, 0)),
        pl.BlockSpec((bb, L), lambda i: (i, 0)),
        pl.BlockSpec((bb, 1), lambda i: (i, 0)),
        pl.BlockSpec((bb, 1), lambda i: (i, 0)),
        pl.BlockSpec((D, 128), full0),
        pl.BlockSpec((TE, 128), full0),
        pl.BlockSpec((1, 128), full0),
        pl.BlockSpec((128, 128), full0),
        pl.BlockSpec((128, 192), full0),
        pl.BlockSpec((1, 192), full0),
        pl.BlockSpec((10, 16), full0),
        pl.BlockSpec((10, 16), full0),
        pl.BlockSpec((352, 352), full0),
        pl.BlockSpec((1, 352), full0),
        pl.BlockSpec((352, 352), full0),
        pl.BlockSpec((1, 352), full0),
        pl.BlockSpec((352, 256), full0),
        pl.BlockSpec((1, 256), full0),
        pl.BlockSpec((256, 128), full0),
        pl.BlockSpec((1, 128), full0),
        pl.BlockSpec((352, 1), full0),
        pl.BlockSpec((128, 1), full0),
        pl.BlockSpec((1, 1), full0),
    ]
    return pl.pallas_call(
        functools.partial(_tc_body, bb=bb),
        grid=grid,
        in_specs=specs,
        out_specs=pl.BlockSpec((bb, 1), lambda i: (i, 0)),
        out_shape=jax.ShapeDtypeStruct((B, 1), jnp.float32),
    )(fro_seq, te_seq, fro_tgt, te_tgt, item_seqs, likes2, views2,
      wfa, wfb, bf2, wat, wtp, btp2, ltab, vtab,
      wc1, bc12, wc2, bc22, wd1, bd12, wd2, bd22, woa, wob, bo2)


def kernel(item_seqs, item_ids, likes_levels, views_levels, frozen_table,
           item_tags, tag_table, W_fuse, b_fuse, W_attn, W_tproj, b_tproj,
           likes_table, views_table, W_c1, b_c1, W_c2, b_c2, W_d1, b_d1,
           W_d2, b_d2, W_out, b_out):
    seq_ids = item_seqs.reshape(-1).astype(jnp.int32)
    tgt_ids = item_ids.astype(jnp.int32)
    tags16 = jnp.pad(item_tags.astype(jnp.int32), ((0, 0), (0, 11)))
    fro_b = frozen_table.astype(jnp.bfloat16)

    fro_all, te_all = _sc_gather(seq_ids, tgt_ids, fro_b, tags16, tag_table)

    wfa = W_fuse[:D]
    wfb = W_fuse[D:]
    wat = W_attn.T
    out = _tc_call(
        fro_all, te_all, fro_all, te_all, item_seqs.astype(jnp.int32),
        likes_levels.astype(jnp.int32).reshape(B, 1),
        views_levels.astype(jnp.int32).reshape(B, 1),
        wfa, wfb, b_fuse.reshape(1, 128), wat, W_tproj,
        b_tproj.reshape(1, 192), likes_table, views_table,
        W_c1, b_c1.reshape(1, 352), W_c2, b_c2.reshape(1, 352),
        W_d1, b_d1.reshape(1, 256), W_d2, b_d2.reshape(1, 128),
        W_out[:352], W_out[352:], b_out.reshape(1, 1))
    return out[:, 0]


# speedup vs baseline: 3.0032x; 1.6809x over previous
import functools
import math

import jax
import jax.numpy as jnp
from jax import lax
from jax.experimental import pallas as pl
from jax.experimental.pallas import tpu as pltpu
from jax.experimental.pallas import tpu_sc as plsc

V = 100000
T = 10000
B = 4096
L = 50
D = 128
TE = 16
K = 16

NW = 32
CH = 128
NIDS = B * (L + 1)
N_PER = NIDS // NW
NCH = N_PER // CH
FW = D // 2
NBUF = 4


def _make_sc_body(nb, ch):
    nids = nb * (L + 1)
    n_per = nids // NW
    nch = n_per // ch
    bl = nb * L

    def body(seq_ids, tgt_ids, fro_tab, tags16, tag_tab,
             fro_all, te_all, *scr):
        bufs = [scr[i * 6:(i + 1) * 6] for i in range(NBUF)]
        sems = scr[NBUF * 6:]
        wid = lax.axis_index("s") * 2 + lax.axis_index("c")
        wbase = wid * n_per
        lanes = lax.iota(jnp.int32, 16)
        patp = lanes * ch
        mask5 = lanes < 5

        def sem4(k, j):
            return sems[k * 4 + j]

        def issue(c, k):
            idx_v, tg_v, slot_v, fro_v, tr_v, te_v = bufs[k]
            static = isinstance(c, int)
            base = wbase + (min(c, nch - 1) if static else c) * ch

            def drain():
                pltpu.make_async_copy(fro_v, fro_all.at[pl.ds(base, ch)],
                                      sem4(k, 3)).wait()
                pltpu.make_async_copy(te_v, te_all.at[pl.ds(base, ch)],
                                      sem4(k, 3)).wait()

            def load():
                @pl.when(base < bl)
                def _():
                    pltpu.sync_copy(seq_ids.at[pl.ds(base, ch)], idx_v)

                @pl.when(base >= bl)
                def _():
                    pltpu.sync_copy(tgt_ids.at[pl.ds(base - bl, ch)], idx_v)

                pltpu.async_copy(fro_tab.at[idx_v], fro_v, sem4(k, 0))
                pltpu.async_copy(tags16.at[idx_v], tg_v, sem4(k, 1))

            if static:
                if c >= NBUF:
                    drain()
                if c < nch:
                    load()
            else:
                pl.when(c >= NBUF)(drain)
                load()

        def process(c, k):
            idx_v, tg_v, slot_v, fro_v, tr_v, te_v = bufs[k]
            base = wbase + c * ch
            pltpu.make_async_copy(tags16.at[idx_v], tg_v, sem4(k, 1)).wait()

            def tr_body(i, _):
                plsc.store_scatter(slot_v, [patp + i], tg_v[i, :], mask=mask5)
                return 0

            lax.fori_loop(0, ch, tr_body, 0)
            h_t = [pltpu.async_copy(tag_tab.at[slot_v.at[pl.ds(s * ch, ch)]],
                                    tr_v.at[s], sem4(k, 2))
                   for s in range(5)]
            for h in h_t:
                h.wait()

            def sum_body(i, _):
                te_v[i, :] = (tr_v[0, i, :] + tr_v[1, i, :] + tr_v[2, i, :]
                              + tr_v[3, i, :] + tr_v[4, i, :])
                return 0

            lax.fori_loop(0, ch, sum_body, 0)
            pltpu.make_async_copy(fro_tab.at[idx_v], fro_v, sem4(k, 0)).wait()
            pltpu.async_copy(fro_v, fro_all.at[pl.ds(base, ch)], sem4(k, 3))
            pltpu.async_copy(te_v, te_all.at[pl.ds(base, ch)], sem4(k, 3))

        issue(0, 0)
        issue(1, 1)

        def step(c, k):
            issue(c + 2, (k + 2) % NBUF)
            process(c, k)

        def loop_body(g, _):
            c = g * NBUF
            for k in range(NBUF):
                step(c + k, k)
            return 0

        nfull = (nch // NBUF) * NBUF
        lax.fori_loop(0, nch // NBUF, loop_body, 0)
        for k in range(nch - nfull):
            step(nfull + k, k)
        for c in range(nch - 2, nch):
            k = c % NBUF
            idx_v, tg_v, slot_v, fro_v, tr_v, te_v = bufs[k]
            base = wbase + c * ch
            pltpu.make_async_copy(fro_v, fro_all.at[pl.ds(base, ch)],
                                  sem4(k, 3)).wait()
            pltpu.make_async_copy(te_v, te_all.at[pl.ds(base, ch)],
                                  sem4(k, 3)).wait()

    return body


def _sc_gather(seq_ids, tgt_ids, fro_tab, tags16, tag_tab, nb, ch):
    nids = nb * (L + 1)
    mesh = plsc.VectorSubcoreMesh(core_axis_name="c", subcore_axis_name="s")
    scratch = []
    for _ in range(NBUF):
        scratch += [
            pltpu.VMEM((ch,), jnp.int32),
            pltpu.VMEM((ch, 16), jnp.int32),
            pltpu.VMEM((5 * ch,), jnp.int32),
            pltpu.VMEM((ch, D), jnp.float32),
            pltpu.VMEM((5, ch, TE), jnp.float32),
            pltpu.VMEM((ch, TE), jnp.float32),
        ]
    scratch += [pltpu.SemaphoreType.DMA] * (NBUF * 4)
    f = pl.kernel(
        _make_sc_body(nb, ch),
        out_type=(
            jax.ShapeDtypeStruct((nids, D), jnp.float32),
            jax.ShapeDtypeStruct((nids, TE), jnp.float32),
        ),
        mesh=mesh,
        compiler_params=pltpu.CompilerParams(use_tc_tiling_on_sc=False,
                                             needs_layout_passes=False),
        scratch_types=scratch,
    )
    return f(seq_ids, tgt_ids, fro_tab, tags16, tag_tab)


def _tc_body(fro_s_ref, te_s_ref, fro_t_ref, te_t_ref, seqid_ref,
             likes_ref, views_ref,
             wfa_ref, wfb_ref, bf_ref, wat_ref, wtp_ref, btp_ref,
             ltab_ref, vtab_ref,
             wc1_ref, bc1_ref, wc2_ref, bc2_ref,
             wd1_ref, bd1_ref, wd2_ref, bd2_ref,
             woa_ref, wob_ref, bo_ref, out_ref, *, bb):
    def bfc(x):
        return x.astype(jnp.bfloat16)

    def dotf(a, b):
        return jnp.dot(bfc(a), bfc(b), preferred_element_type=jnp.float32)

    wfa = wfa_ref[...]
    wfb = wfb_ref[...]
    bf = bf_ref[...]

    s_in = fro_s_ref[...]
    te_s = te_s_ref[...] * 0.2
    s_emb = jnp.tanh(dotf(s_in, wfa) + dotf(te_s, wfb) + bf)
    t_emb = jnp.tanh(dotf(fro_t_ref[...], wfa)
                     + dotf(te_t_ref[...] * 0.2, wfb) + bf)

    q = dotf(t_emb, wat_ref[...])
    s3 = s_emb.reshape(bb, L, 128)
    s3b = bfc(s3).astype(jnp.float32)
    scores = jnp.sum(s3b * bfc(q)[:, None, :].astype(jnp.float32), axis=-1)
    scores = scores * (1.0 / math.sqrt(128.0))
    scores = jnp.where(seqid_ref[...] == 0, -1e9, scores)

    lp = lax.broadcasted_iota(jnp.int32, (bb, L, L), 1)
    ll = lax.broadcasted_iota(jnp.int32, (bb, L, L), 2)
    a_lp = scores[:, :, None]
    a_l = scores[:, None, :]
    rank = jnp.sum(
        (a_lp > a_l).astype(jnp.int32)
        + jnp.logical_and(a_lp == a_l, lp < ll).astype(jnp.int32), axis=1)
    sel = rank < K

    m = jnp.max(scores, axis=1, keepdims=True)
    e = jnp.where(sel, jnp.exp(scores - m), 0.0)
    w = e / jnp.sum(e, axis=1, keepdims=True)
    wb = bfc(w).astype(jnp.float32)
    S_o = jnp.sum(wb[:, :, None] * s3b, axis=1)

    t_proj = dotf(t_emb, wtp_ref[...]) + btp_ref[...]
    lv_iota = lax.broadcasted_iota(jnp.int32, (bb, 10), 1)
    l_oh = (likes_ref[...] == lv_iota).astype(jnp.float32)
    v_oh = (views_ref[...] == lv_iota).astype(jnp.float32)
    le = jnp.sum(l_oh[:, :, None] * ltab_ref[...][None, :, :], axis=1)
    ve = jnp.sum(v_oh[:, :, None] * vtab_ref[...][None, :, :], axis=1)

    x0 = jnp.concatenate([S_o, t_proj, le, ve], axis=-1)
    x1 = x0 * (dotf(x0, wc1_ref[...]) + bc1_ref[...]) + x0
    x2 = x0 * (dotf(x1, wc2_ref[...]) + bc2_ref[...]) + x1
    deep = jnp.maximum(dotf(x0, wd1_ref[...]) + bd1_ref[...], 0.0)
    deep = jnp.maximum(dotf(deep, wd2_ref[...]) + bd2_ref[...], 0.0)
    out_ref[...] = (dotf(x2, woa_ref[...]) + dotf(deep, wob_ref[...])
                    + bo_ref[...])


def _tc_call(nb, fro_all, te_all, item_seqs, likes2, views2,
             wfa, wfb, bf2, wat, wtp, btp2, ltab, vtab,
             wc1, bc12, wc2, bc22, wd1, bd12, wd2, bd22, woa, wob, bo2):
    bb = 256
    grid = (nb // bb,)
    tb = nb * L // bb

    full0 = lambda i: (0, 0)
    specs = [
        pl.BlockSpec((bb * L, D), lambda i: (i, 0)),
        pl.BlockSpec((bb * L, TE), lambda i: (i, 0)),
        pl.BlockSpec((bb, D), lambda i: (tb + i, 0)),
        pl.BlockSpec((bb, TE), lambda i: (tb + i, 0)),
        pl.BlockSpec((bb, L), lambda i: (i, 0)),
        pl.BlockSpec((bb, 1), lambda i: (i, 0)),
        pl.BlockSpec((bb, 1), lambda i: (i, 0)),
        pl.BlockSpec((D, 128), full0),
        pl.BlockSpec((TE, 128), full0),
        pl.BlockSpec((1, 128), full0),
        pl.BlockSpec((128, 128), full0),
        pl.BlockSpec((128, 192), full0),
        pl.BlockSpec((1, 192), full0),
        pl.BlockSpec((10, 16), full0),
        pl.BlockSpec((10, 16), full0),
        pl.BlockSpec((352, 352), full0),
        pl.BlockSpec((1, 352), full0),
        pl.BlockSpec((352, 352), full0),
        pl.BlockSpec((1, 352), full0),
        pl.BlockSpec((352, 256), full0),
        pl.BlockSpec((1, 256), full0),
        pl.BlockSpec((256, 128), full0),
        pl.BlockSpec((1, 128), full0),
        pl.BlockSpec((352, 1), full0),
        pl.BlockSpec((128, 1), full0),
        pl.BlockSpec((1, 1), full0),
    ]
    return pl.pallas_call(
        functools.partial(_tc_body, bb=bb),
        grid=grid,
        in_specs=specs,
        out_specs=pl.BlockSpec((bb, 1), lambda i: (i, 0)),
        out_shape=jax.ShapeDtypeStruct((nb, 1), jnp.float32),
    )(fro_all, te_all, fro_all, te_all, item_seqs, likes2, views2,
      wfa, wfb, bf2, wat, wtp, btp2, ltab, vtab,
      wc1, bc12, wc2, bc22, wd1, bd12, wd2, bd22, woa, wob, bo2)


def kernel(item_seqs, item_ids, likes_levels, views_levels, frozen_table,
           item_tags, tag_table, W_fuse, b_fuse, W_attn, W_tproj, b_tproj,
           likes_table, views_table, W_c1, b_c1, W_c2, b_c2, W_d1, b_d1,
           W_d2, b_d2, W_out, b_out):
    tags16 = jnp.pad(item_tags.astype(jnp.int32), ((0, 0), (0, 11)))
    iseqs = item_seqs.astype(jnp.int32)
    tgts = item_ids.astype(jnp.int32)
    likes2 = likes_levels.astype(jnp.int32).reshape(B, 1)
    views2 = views_levels.astype(jnp.int32).reshape(B, 1)

    wfa = W_fuse[:D]
    wfb = W_fuse[D:]
    wat = W_attn.T
    weights = (wfa, wfb, b_fuse.reshape(1, 128), wat, W_tproj,
               b_tproj.reshape(1, 192), likes_table, views_table,
               W_c1, b_c1.reshape(1, 352), W_c2, b_c2.reshape(1, 352),
               W_d1, b_d1.reshape(1, 256), W_d2, b_d2.reshape(1, 128),
               W_out[:352], W_out[352:], b_out.reshape(1, 1))

    nb = B // 2
    ch = 64
    outs = []
    gathered = []
    for h in range(2):
        sl = slice(h * nb, (h + 1) * nb)
        gathered.append(_sc_gather(iseqs[sl].reshape(-1), tgts[sl],
                                   frozen_table, tags16, tag_table, nb, ch))
    for h in range(2):
        sl = slice(h * nb, (h + 1) * nb)
        fro_all, te_all = gathered[h]
        outs.append(_tc_call(nb, fro_all, te_all, iseqs[sl],
                             likes2[sl], views2[sl], *weights))
    return jnp.concatenate(outs)[:, 0]
